# Initial kernel scaffold; baseline (speedup 1.0000x reference)
import jax
import jax.numpy as jnp
from jax.experimental import pallas as pl


def kernel(hidden_states, w_gate, w1, w3, w2):
    raise NotImplementedError("write your pallas kernel here")


# speedup vs baseline: 1.9198x; 1.9198x over previous
import functools

import jax
import jax.numpy as jnp
from jax.experimental import pallas as pl
from jax.experimental.pallas import tpu as pltpu

E = 8
TOPK = 2
H = 1024
FF = 4096
T = 256
FFB = 1024
NF = FF // FFB


def _moe_body(x_ref, xb_ref, wg_ref, w1_ref, w3_ref, w2_ref, out_ref, comb_ref):
    e = pl.program_id(0)
    f = pl.program_id(1)

    @pl.when(jnp.logical_and(e == 0, f == 0))
    def _router():
        x = x_ref[...]
        logits = jnp.dot(x, wg_ref[...], preferred_element_type=jnp.float32)
        m = jnp.max(logits, axis=-1, keepdims=True)
        ex = jnp.exp(logits - m)
        probs = ex / jnp.sum(ex, axis=-1, keepdims=True)
        lane = jax.lax.broadcasted_iota(jnp.int32, (T, E), 1)
        m1 = jnp.max(probs, axis=-1, keepdims=True)
        i1 = jnp.min(jnp.where(probs == m1, lane, E), axis=-1, keepdims=True)
        probs2 = jnp.where(lane == i1, -jnp.inf, probs)
        m2 = jnp.max(probs2, axis=-1, keepdims=True)
        i2 = jnp.min(jnp.where(probs2 == m2, lane, E), axis=-1, keepdims=True)
        comb = jnp.where(lane == i1, m1, 0.0) + jnp.where(lane == i2, m2, 0.0)
        comb_ref[...] = comb / (m1 + m2)
        out_ref[...] = jnp.zeros((T, H), jnp.float32)

    xb = xb_ref[...]
    w1b = w1_ref[0].astype(jnp.bfloat16)
    w3b = w3_ref[0].astype(jnp.bfloat16)
    g = jnp.dot(xb, w1b, preferred_element_type=jnp.float32)
    u = jnp.dot(xb, w3b, preferred_element_type=jnp.float32)
    h = (g * jax.nn.sigmoid(g)) * u
    w2b = w2_ref[0].astype(jnp.bfloat16)
    y = jnp.dot(h.astype(jnp.bfloat16), w2b, preferred_element_type=jnp.float32)
    lane = jax.lax.broadcasted_iota(jnp.int32, (T, E), 1)
    comb_col = jnp.sum(jnp.where(lane == e, comb_ref[...], 0.0), axis=-1,
                       keepdims=True)
    out_ref[...] += comb_col * y


@functools.partial(jax.jit, static_argnames=("interpret",))
def kernel(hidden_states, w_gate, w1, w3, w2, interpret=False):
    xb = hidden_states.astype(jnp.bfloat16)
    return pl.pallas_call(
        _moe_body,
        grid=(E, NF),
        in_specs=[
            pl.BlockSpec((T, H), lambda e, f: (0, 0)),
            pl.BlockSpec((T, H), lambda e, f: (0, 0)),
            pl.BlockSpec((H, E), lambda e, f: (0, 0)),
            pl.BlockSpec((1, H, FFB), lambda e, f: (e, 0, f)),
            pl.BlockSpec((1, H, FFB), lambda e, f: (e, 0, f)),
            pl.BlockSpec((1, FFB, H), lambda e, f: (e, f, 0)),
        ],
        out_specs=pl.BlockSpec((T, H), lambda e, f: (0, 0)),
        out_shape=jax.ShapeDtypeStruct((T, H), jnp.float32),
        scratch_shapes=[pltpu.VMEM((T, E), jnp.float32)],
        compiler_params=pltpu.CompilerParams(
            dimension_semantics=("arbitrary", "arbitrary")),
        interpret=interpret,
    )(hidden_states, xb, w_gate, w1, w3, w2)
